# baseline (speedup 1.0000x reference)
import functools

import jax
import jax.numpy as jnp
from jax import lax
from jax.experimental import pallas as pl
from jax.experimental.pallas import tpu as pltpu
from jax.experimental.pallas import tpu_sc as plsc

VOCAB = 1000000
EMBED = 64
MAX_SEQ = 200
BATCH = 4096

NUM_CORES = 2
NUM_SUBCORES = 16
NUM_WORKERS = NUM_CORES * NUM_SUBCORES
BW = BATCH // NUM_WORKERS
LANES = 16
DC = EMBED // LANES
G = 2
NMAC = MAX_SEQ // G

_mesh = plsc.VectorSubcoreMesh(core_axis_name="c", subcore_axis_name="s")


@functools.partial(
    pl.kernel,
    mesh=_mesh,
    compiler_params=pltpu.CompilerParams(
        use_tc_tiling_on_sc=False, needs_layout_passes=False
    ),
    out_type=jax.ShapeDtypeStruct((MAX_SEQ, 8, 32, 8, 128), jnp.float32),
    scratch_types=[
        pltpu.VMEM((MAX_SEQ, EMBED), jnp.float32),
        pltpu.VMEM((G, BW), jnp.int32),
        pltpu.VMEM((G, BW), jnp.int32),
        pltpu.VMEM((G * BW, 2 * EMBED), jnp.float32),
        pltpu.VMEM((G * BW, 2 * EMBED), jnp.float32),
        pltpu.VMEM((8, 8, 129), jnp.float32),
        pltpu.VMEM((8, 8, 129), jnp.float32),
        pltpu.SemaphoreType.DMA,
        pltpu.SemaphoreType.DMA,
        pltpu.SemaphoreType.DMA,
        pltpu.SemaphoreType.DMA,
    ],
)
def _embed(x5_hbm, tok_hbm, pos_hbm, out_hbm,
           pos_v, ib_a, ib_b, g_a, g_b, t_a, t_b,
           gs_a, gs_b, ss_a, ss_b):
    w = lax.axis_index("s") * NUM_CORES + lax.axis_index("c")
    pltpu.sync_copy(pos_hbm, pos_v)

    lanes = jnp.arange(16, dtype=jnp.int32)
    tr_vecs = [(lanes + c * 16) >> 3 for c in range(DC)]
    r_vecs = [(lanes + c * 16) & 7 for c in range(DC)]
    zeros16 = jnp.zeros((16,), jnp.int32)

    def load_idx(m, ib):
        s0 = G * m
        pltpu.sync_copy(x5_hbm.at[s0 >> 3, w, pl.ds(s0 & 7, G)], ib)

    def fire_gathers(g_v, ib, sem):
        for j in range(G):
            pltpu.async_copy(
                tok_hbm.at[ib.at[j]], g_v.at[pl.ds(j * BW, BW)], sem
            )

    def drain_gathers(g_v, ib, sem):
        for j in range(G):
            pltpu.make_async_copy(
                tok_hbm.at[ib.at[j]], g_v.at[pl.ds(j * BW, BW)], sem
            ).wait()

    def transpose_add(s, g_v, j, t_v):
        pvec = [pos_v[s, pl.ds(c * 16, 16)] for c in range(DC)]
        base = j * BW

        def b_body(b, col):
            row = base + b
            loads = [g_v[row, pl.ds(c * 16, 16)] for c in range(DC)]
            sums = [loads[c] + pvec[c] for c in range(DC)]
            for c in range(DC):
                plsc.store_scatter(t_v, [tr_vecs[c], r_vecs[c], col], sums[c])
            return col + 1

        lax.fori_loop(0, BW, b_body, zeros16, unroll=16)

    def fire_store(s, t_v, sem):
        pltpu.async_copy(t_v.at[:, :, pl.ds(0, 128)], out_hbm.at[s, :, w], sem)

    def wait_store(t_v, sem):
        pltpu.make_async_copy(t_v.at[:, :, pl.ds(0, 128)], out_hbm.at[0, :, w], sem).wait()

    tset = (t_a, t_b)
    sset = (ss_a, ss_b)

    def process_macro(m, g_v, skip_first_waits=False):
        for j in range(G):
            t_v, sem = tset[j % 2], sset[j % 2]
            if not (skip_first_waits and j < 2):
                wait_store(t_v, sem)
            s = G * m + j
            transpose_add(s, g_v, j, t_v)
            fire_store(s, t_v, sem)

    load_idx(0, ib_a)
    fire_gathers(g_a, ib_a, gs_a)
    load_idx(1, ib_b)
    fire_gathers(g_b, ib_b, gs_b)
    drain_gathers(g_a, ib_a, gs_a)
    process_macro(0, g_a, skip_first_waits=True)
    load_idx(2, ib_a)
    fire_gathers(g_a, ib_a, gs_a)
    drain_gathers(g_b, ib_b, gs_b)
    process_macro(1, g_b)

    npairs = NMAC // 2

    def pair_body(p, carry):
        mo = 2 * p + 1
        load_idx(mo, ib_b)
        fire_gathers(g_b, ib_b, gs_b)
        drain_gathers(g_a, ib_a, gs_a)
        process_macro(2 * p, g_a)

        @pl.when(p < npairs - 1)
        def _():
            load_idx(2 * p + 2, ib_a)
            fire_gathers(g_a, ib_a, gs_a)

        drain_gathers(g_b, ib_b, gs_b)
        process_macro(mo, g_b)
        return carry

    lax.fori_loop(1, npairs, pair_body, 0)
    wait_store(t_a, ss_a)
    wait_store(t_b, ss_b)


def kernel(x, token_table, pos_table):
    x5 = x.T.reshape(MAX_SEQ // 8, 8, 32, 128).transpose(0, 2, 1, 3).astype(jnp.int32)
    tok_pad = jnp.pad(token_table, ((0, 0), (0, EMBED)))
    out5 = _embed(x5, tok_pad, pos_table)
    out = out5.transpose(0, 1, 3, 2, 4).reshape(MAX_SEQ, EMBED, BATCH)
    return out.transpose(2, 0, 1)


# speedup vs baseline: 1.0225x; 1.0225x over previous
import functools

import jax
import jax.numpy as jnp
from jax import lax
from jax.experimental import pallas as pl
from jax.experimental.pallas import tpu as pltpu
from jax.experimental.pallas import tpu_sc as plsc

VOCAB = 1000000
EMBED = 64
MAX_SEQ = 200
BATCH = 4096

NUM_CORES = 2
NUM_SUBCORES = 16
NUM_WORKERS = NUM_CORES * NUM_SUBCORES
BW = BATCH // NUM_WORKERS
LANES = 16
DC = EMBED // LANES
G = 2
NMAC = MAX_SEQ // G

_mesh = plsc.VectorSubcoreMesh(core_axis_name="c", subcore_axis_name="s")


@functools.partial(
    pl.kernel,
    mesh=_mesh,
    compiler_params=pltpu.CompilerParams(
        use_tc_tiling_on_sc=False, needs_layout_passes=False
    ),
    out_type=jax.ShapeDtypeStruct((MAX_SEQ, 8, 32, 8, 128), jnp.float32),
    scratch_types=[
        pltpu.VMEM((MAX_SEQ, EMBED), jnp.float32),
        pltpu.VMEM((G, BW), jnp.int32),
        pltpu.VMEM((G, BW), jnp.int32),
        pltpu.VMEM((G * BW, 2 * EMBED), jnp.float32),
        pltpu.VMEM((G * BW, 2 * EMBED), jnp.float32),
        pltpu.VMEM((8, 8, 129), jnp.float32),
        pltpu.VMEM((8, 8, 129), jnp.float32),
        pltpu.SemaphoreType.DMA,
        pltpu.SemaphoreType.DMA,
        pltpu.SemaphoreType.DMA,
        pltpu.SemaphoreType.DMA,
    ],
)
def _embed(x5_hbm, tok_hbm, pos_hbm, out_hbm,
           pos_v, ib_a, ib_b, g_a, g_b, t_a, t_b,
           gs_a, gs_b, ss_a, ss_b):
    w = lax.axis_index("s") * NUM_CORES + lax.axis_index("c")
    pltpu.sync_copy(pos_hbm, pos_v)

    lanes = jnp.arange(16, dtype=jnp.int32)
    tr_vecs = [(lanes + c * 16) >> 3 for c in range(DC)]
    r_vecs = [(lanes + c * 16) & 7 for c in range(DC)]
    zeros16 = jnp.zeros((16,), jnp.int32)

    def load_idx(m, ib):
        s0 = G * m
        pltpu.sync_copy(x5_hbm.at[s0 >> 3, w, pl.ds(s0 & 7, G)], ib)

    def fire_gathers(g_v, ib, sem):
        for j in range(G):
            pltpu.async_copy(
                tok_hbm.at[ib.at[j]], g_v.at[pl.ds(j * BW, BW)], sem
            )

    def drain_gathers(g_v, ib, sem):
        for j in range(G):
            pltpu.make_async_copy(
                tok_hbm.at[ib.at[j]], g_v.at[pl.ds(j * BW, BW)], sem
            ).wait()

    def transpose_add(s, g_v, j, t_v):
        pvec = [pos_v[s, pl.ds(c * 16, 16)] for c in range(DC)]
        base = j * BW

        def b_body(b, col):
            row = base + b
            loads = [g_v[row, pl.ds(c * 16, 16)] for c in range(DC)]
            sums = [loads[c] + pvec[c] for c in range(DC)]
            for c in range(DC):
                plsc.store_scatter(t_v, [tr_vecs[c], r_vecs[c], col], sums[c])
            return col + 1

        lax.fori_loop(0, BW, b_body, zeros16, unroll=8)

    def fire_store(s, t_v, sem):
        pltpu.async_copy(t_v.at[:, :, pl.ds(0, 128)], out_hbm.at[s, :, w], sem)

    def wait_store(t_v, sem):
        pltpu.make_async_copy(t_v.at[:, :, pl.ds(0, 128)], out_hbm.at[0, :, w], sem).wait()

    tset = (t_a, t_b)
    sset = (ss_a, ss_b)

    def process_macro(m, g_v, skip_first_waits=False):
        for j in range(G):
            t_v, sem = tset[j % 2], sset[j % 2]
            if not (skip_first_waits and j < 2):
                wait_store(t_v, sem)
            s = G * m + j
            transpose_add(s, g_v, j, t_v)
            fire_store(s, t_v, sem)

    load_idx(0, ib_a)
    fire_gathers(g_a, ib_a, gs_a)
    load_idx(1, ib_b)
    fire_gathers(g_b, ib_b, gs_b)
    drain_gathers(g_a, ib_a, gs_a)
    process_macro(0, g_a, skip_first_waits=True)
    load_idx(2, ib_a)
    fire_gathers(g_a, ib_a, gs_a)
    drain_gathers(g_b, ib_b, gs_b)
    process_macro(1, g_b)

    npairs = NMAC // 2

    def pair_body(p, carry):
        mo = 2 * p + 1
        load_idx(mo, ib_b)
        fire_gathers(g_b, ib_b, gs_b)
        drain_gathers(g_a, ib_a, gs_a)
        process_macro(2 * p, g_a)

        @pl.when(p < npairs - 1)
        def _():
            load_idx(2 * p + 2, ib_a)
            fire_gathers(g_a, ib_a, gs_a)

        drain_gathers(g_b, ib_b, gs_b)
        process_macro(mo, g_b)
        return carry

    lax.fori_loop(1, npairs, pair_body, 0)
    wait_store(t_a, ss_a)
    wait_store(t_b, ss_b)


def kernel(x, token_table, pos_table):
    x5 = x.T.reshape(MAX_SEQ // 8, 8, 32, 128).transpose(0, 2, 1, 3).astype(jnp.int32)
    tok_pad = jnp.pad(token_table, ((0, 0), (0, EMBED)))
    out5 = _embed(x5, tok_pad, pos_table)
    out = out5.transpose(0, 1, 3, 2, 4).reshape(MAX_SEQ, EMBED, BATCH)
    return out.transpose(2, 0, 1)
